# baseline (speedup 1.0000x reference)
import jax
import jax.numpy as jnp
from jax import lax
from jax.experimental import pallas as pl
from jax.experimental.pallas import tpu as pltpu
from jax.experimental.pallas import tpu_sc as plsc

BATCH = 1024
SEQ = 200
D = 128
LANES = 16
NUM_WORKERS = 32
SEQ_PER_W = BATCH // NUM_WORKERS
SPLIT = 104
NBUF = 3
CHUNKS_LO = ((0, 56), (56, 48))
CHUNKS_HI = ((104, 48), (152, 48))


def _issue_gather(tgt_hbm, idx_all, s, rows, sem_lo, sem_hi):
    pltpu.async_copy(
        tgt_hbm.at[idx_all.at[pl.ds(s * SEQ, SPLIT)]], rows.at[pl.ds(0, SPLIT)], sem_lo
    )
    pltpu.async_copy(
        tgt_hbm.at[idx_all.at[pl.ds(s * SEQ + SPLIT, SEQ - SPLIT)]],
        rows.at[pl.ds(SPLIT, SEQ - SPLIT)],
        sem_hi,
    )


def _emb_body(
    x_hbm, tgt_hbm, pos_hbm, out_hbm,
    idx_all, rows0, rows1, rows2, pos_v,
    glo0, glo1, glo2, ghi0, ghi1, ghi2, wsem0, wsem1, wsem2,
):
    info = plsc.get_sparse_core_info()
    wid = lax.axis_index("s") * info.num_cores + lax.axis_index("c")
    base = wid * SEQ_PER_W

    pltpu.sync_copy(pos_hbm, pos_v)
    pltpu.sync_copy(x_hbm.at[pl.ds(base * SEQ, SEQ_PER_W * SEQ)], idx_all)

    rows = (rows0, rows1, rows2)
    glo = (glo0, glo1, glo2)
    ghi = (ghi0, ghi1, ghi2)
    wsem = (wsem0, wsem1, wsem2)

    def drain(p, sem, n_rows):
        pltpu.make_async_copy(out_hbm.at[0].at[pl.ds(0, n_rows)],
                              rows[p].at[pl.ds(0, n_rows)], sem).wait()

    def drain_write(p):
        pltpu.make_async_copy(rows[p], out_hbm.at[0], wsem[p]).wait()

    def add_write_chunks(p, s_eff, chunks):
        for off, cnt in chunks:
            @plsc.parallel_loop(off, off + cnt, 1, unroll=8)
            def add_row(r):
                for c in range(D // LANES):
                    sl = pl.ds(c * LANES, LANES)
                    plsc.addupdate(rows[p].at[r, sl], pos_v[r, sl])

            pltpu.async_copy(
                rows[p].at[pl.ds(off, cnt)],
                out_hbm.at[base + s_eff].at[pl.ds(off, cnt)],
                wsem[p],
            )

    def process(p, s_eff):
        drain(p, glo[p], SPLIT)
        add_write_chunks(p, s_eff, CHUNKS_LO)
        drain(p, ghi[p], SEQ - SPLIT)
        add_write_chunks(p, s_eff, CHUNKS_HI)

    _issue_gather(tgt_hbm, idx_all, 0, rows[0], glo[0], ghi[0])

    @pl.loop(0, SEQ_PER_W - 2, step=NBUF)
    def per_triple(s):
        for p in range(NBUF):
            s_eff = s + p

            @pl.when(s_eff >= 1)
            def _():
                drain_write((p + 2) % NBUF)

            q = (p + 1) % NBUF
            _issue_gather(tgt_hbm, idx_all, s_eff + 1, rows[q], glo[q], ghi[q])
            process(p, s_eff)

    drain_write(2)
    _issue_gather(tgt_hbm, idx_all, SEQ_PER_W - 1, rows[1], glo[1], ghi[1])
    process(0, SEQ_PER_W - 2)
    process(1, SEQ_PER_W - 1)
    drain_write(0)
    drain_write(1)


def kernel(X_input, tgt_table, pos_table):
    mesh = plsc.VectorSubcoreMesh(core_axis_name="c", subcore_axis_name="s")
    run = pl.kernel(
        _emb_body,
        out_type=jax.ShapeDtypeStruct((BATCH, SEQ, D), jnp.float32),
        mesh=mesh,
        scratch_types=[
            pltpu.VMEM((SEQ_PER_W * SEQ,), jnp.int32),
            pltpu.VMEM((SEQ, D), jnp.float32),
            pltpu.VMEM((SEQ, D), jnp.float32),
            pltpu.VMEM((SEQ, D), jnp.float32),
            pltpu.VMEM((SEQ, D), jnp.float32),
            pltpu.SemaphoreType.DMA,
            pltpu.SemaphoreType.DMA,
            pltpu.SemaphoreType.DMA,
            pltpu.SemaphoreType.DMA,
            pltpu.SemaphoreType.DMA,
            pltpu.SemaphoreType.DMA,
            pltpu.SemaphoreType.DMA,
            pltpu.SemaphoreType.DMA,
            pltpu.SemaphoreType.DMA,
        ],
    )
    return run(X_input.reshape(-1), tgt_table, pos_table)


# speedup vs baseline: 1.1075x; 1.1075x over previous
import jax
import jax.numpy as jnp
from jax import lax
from jax.experimental import pallas as pl
from jax.experimental.pallas import tpu as pltpu
from jax.experimental.pallas import tpu_sc as plsc

BATCH = 1024
SEQ = 200
D = 128
LANES = 16
NUM_WORKERS = 32
SEQ_PER_W = BATCH // NUM_WORKERS
SPLIT = 104
NBUF = 3
CHUNKS_LO = ((0, SPLIT),)
CHUNKS_HI = ((SPLIT, SEQ - SPLIT),)


def _issue_gather(tgt_hbm, idx_all, s, rows, sem_lo, sem_hi):
    pltpu.async_copy(
        tgt_hbm.at[idx_all.at[pl.ds(s * SEQ, SPLIT)]], rows.at[pl.ds(0, SPLIT)], sem_lo
    )
    pltpu.async_copy(
        tgt_hbm.at[idx_all.at[pl.ds(s * SEQ + SPLIT, SEQ - SPLIT)]],
        rows.at[pl.ds(SPLIT, SEQ - SPLIT)],
        sem_hi,
    )


def _emb_body(
    x_hbm, tgt_hbm, pos_hbm, out_hbm,
    idx_all, rows0, rows1, rows2, pos_v,
    glo0, glo1, glo2, ghi0, ghi1, ghi2, wsem0, wsem1, wsem2,
):
    info = plsc.get_sparse_core_info()
    wid = lax.axis_index("s") * info.num_cores + lax.axis_index("c")
    base = wid * SEQ_PER_W

    pltpu.sync_copy(pos_hbm, pos_v)
    pltpu.sync_copy(x_hbm.at[pl.ds(base * SEQ, SEQ_PER_W * SEQ)], idx_all)

    rows = (rows0, rows1, rows2)
    glo = (glo0, glo1, glo2)
    ghi = (ghi0, ghi1, ghi2)
    wsem = (wsem0, wsem1, wsem2)

    def drain(p, sem, n_rows):
        pltpu.make_async_copy(out_hbm.at[0].at[pl.ds(0, n_rows)],
                              rows[p].at[pl.ds(0, n_rows)], sem).wait()

    def drain_write(p):
        pltpu.make_async_copy(rows[p], out_hbm.at[0], wsem[p]).wait()

    def add_write_chunks(p, s_eff, chunks):
        for off, cnt in chunks:
            @plsc.parallel_loop(off, off + cnt, 1, unroll=8)
            def add_row(r):
                for c in range(D // LANES):
                    sl = pl.ds(c * LANES, LANES)
                    plsc.addupdate(rows[p].at[r, sl], pos_v[r, sl])

            pltpu.async_copy(
                rows[p].at[pl.ds(off, cnt)],
                out_hbm.at[base + s_eff].at[pl.ds(off, cnt)],
                wsem[p],
            )

    def process(p, s_eff):
        drain(p, glo[p], SPLIT)
        add_write_chunks(p, s_eff, CHUNKS_LO)
        drain(p, ghi[p], SEQ - SPLIT)
        add_write_chunks(p, s_eff, CHUNKS_HI)

    _issue_gather(tgt_hbm, idx_all, 0, rows[0], glo[0], ghi[0])

    @pl.loop(0, SEQ_PER_W - 2, step=NBUF)
    def per_triple(s):
        for p in range(NBUF):
            s_eff = s + p

            @pl.when(s_eff >= 1)
            def _():
                drain_write((p + 2) % NBUF)

            q = (p + 1) % NBUF
            _issue_gather(tgt_hbm, idx_all, s_eff + 1, rows[q], glo[q], ghi[q])
            process(p, s_eff)

    drain_write(2)
    _issue_gather(tgt_hbm, idx_all, SEQ_PER_W - 1, rows[1], glo[1], ghi[1])
    process(0, SEQ_PER_W - 2)
    process(1, SEQ_PER_W - 1)
    drain_write(0)
    drain_write(1)


def kernel(X_input, tgt_table, pos_table):
    mesh = plsc.VectorSubcoreMesh(core_axis_name="c", subcore_axis_name="s")
    run = pl.kernel(
        _emb_body,
        out_type=jax.ShapeDtypeStruct((BATCH, SEQ, D), jnp.float32),
        mesh=mesh,
        scratch_types=[
            pltpu.VMEM((SEQ_PER_W * SEQ,), jnp.int32),
            pltpu.VMEM((SEQ, D), jnp.float32),
            pltpu.VMEM((SEQ, D), jnp.float32),
            pltpu.VMEM((SEQ, D), jnp.float32),
            pltpu.VMEM((SEQ, D), jnp.float32),
            pltpu.SemaphoreType.DMA,
            pltpu.SemaphoreType.DMA,
            pltpu.SemaphoreType.DMA,
            pltpu.SemaphoreType.DMA,
            pltpu.SemaphoreType.DMA,
            pltpu.SemaphoreType.DMA,
            pltpu.SemaphoreType.DMA,
            pltpu.SemaphoreType.DMA,
            pltpu.SemaphoreType.DMA,
        ],
    )
    return run(X_input.reshape(-1), tgt_table, pos_table)
